# baseline (speedup 1.0000x reference)
import functools

import jax
import jax.numpy as jnp
from jax import lax
from jax.experimental import pallas as pl
from jax.experimental.pallas import tpu as pltpu
from jax.experimental.pallas import tpu_sc as plsc

N = 10000
D = 128
E = 320000
NC = 2
NS = 16
NW = NC * NS
CH = 80
NCHUNK0 = 147
NCHUNK1 = 105
EPW0 = NCHUNK0 * CH
EPW1 = NCHUNK1 * CH
EP = (EPW0 + EPW1) * NS
NP = 10240
RPT = NP // NS
ZCH = CH


def _sc_aggregate(h, src, dst):
    mesh = plsc.VectorSubcoreMesh(core_axis_name="c", subcore_axis_name="s")

    @functools.partial(
        pl.kernel,
        mesh=mesh,
        out_type=[
            jax.ShapeDtypeStruct((NP, D), jnp.float32),
            jax.ShapeDtypeStruct((NP, D), jnp.float32),
        ],
        scratch_types=[
            pltpu.VMEM((CH,), jnp.int32),
            pltpu.VMEM((CH,), jnp.int32),
            pltpu.VMEM((CH,), jnp.int32),
            pltpu.VMEM((CH,), jnp.int32),
            pltpu.VMEM((CH,), jnp.int32),
            pltpu.VMEM((CH,), jnp.int32),
            pltpu.VMEM((CH,), jnp.int32),
            pltpu.VMEM((CH,), jnp.int32),
            pltpu.VMEM((CH,), jnp.int32),
            pltpu.VMEM((CH, D), jnp.float32),
            pltpu.VMEM((CH, D), jnp.float32),
            pltpu.VMEM((CH, D), jnp.float32),
            pltpu.VMEM_SHARED((NP, D), jnp.float32),
            pltpu.SemaphoreType.DMA,
            pltpu.SemaphoreType.DMA,
            pltpu.SemaphoreType.DMA,
            pltpu.SemaphoreType.DMA,
            pltpu.SemaphoreType.DMA,
            pltpu.SemaphoreType.DMA,
            pltpu.SemaphoreType.DMA,
            pltpu.SemaphoreType.DMA,
            pltpu.SemaphoreType.DMA,
            pltpu.SemaphoreType.DMA,
            pltpu.SemaphoreType.DMA,
            pltpu.SemaphoreType.DMA,
        ],
    )
    def agg_kernel(h_hbm, src_hbm, dst_hbm, out0, out1,
                   sidx0, sidx1, sidx2, didx0, didx1, didx2,
                   dcp0, dcp1, dcp2, rows0, rows1, rows2, acc,
                   gsem0, gsem1, gsem2, ssem0, ssem1, ssem2,
                   sisem0, sisem1, sisem2, disem0, disem1, disem2):
        cid = lax.axis_index("c")
        sid = lax.axis_index("s")
        wid = cid * NS + sid

        z = jnp.zeros((16,), jnp.float32)

        def zstore(i, _):
            r = i // (D // 16)
            k = i % (D // 16)
            rows0[r, pl.ds(k * 16, 16)] = z
            return 0

        lax.fori_loop(0, ZCH * (D // 16), zstore, 0)

        def zcopy(j, _):
            pltpu.sync_copy(rows0, acc.at[pl.ds(sid * RPT + j * ZCH, ZCH)])
            return 0

        lax.fori_loop(0, RPT // ZCH, zcopy, 0)
        plsc.subcore_barrier()

        nchunk = jnp.where(cid == 0, NCHUNK0, NCHUNK1)
        base0 = jnp.where(cid == 0, sid * EPW0, NS * EPW0 + sid * EPW1)
        slots = (
            (sidx0, didx0, dcp0, rows0, gsem0, ssem0, sisem0, disem0),
            (sidx1, didx1, dcp1, rows1, gsem1, ssem1, sisem1, disem1),
            (sidx2, didx2, dcp2, rows2, gsem2, ssem2, sisem2, disem2),
        )

        def iload(j, s):
            si, di = s[0], s[1]
            pltpu.async_copy(src_hbm.at[pl.ds(base0 + j * CH, CH)], si, s[6])
            pltpu.async_copy(dst_hbm.at[pl.ds(base0 + j * CH, CH)], di, s[7])

        def iwait(j, s):
            si, di = s[0], s[1]
            pltpu.make_async_copy(
                src_hbm.at[pl.ds(base0 + j * CH, CH)], si, s[6]).wait()
            pltpu.make_async_copy(
                dst_hbm.at[pl.ds(base0 + j * CH, CH)], di, s[7]).wait()

        def gstart(s):
            pltpu.async_copy(h_hbm.at[s[0]], s[3], s[4])

        def gwait(s):
            pltpu.make_async_copy(h_hbm.at[s[0]], s[3], s[4]).wait()

        def sstart(s):
            pltpu.async_copy(s[3], acc.at[s[2]], s[5], add=True)

        def swait(s):
            pltpu.make_async_copy(s[3], acc.at[s[2]], s[5]).wait()

        def dcopy(s):
            di, dc = s[1], s[2]
            for k in range(CH // 16):
                dc[pl.ds(k * 16, 16)] = di[pl.ds(k * 16, 16)]

        iload(0, slots[0])
        iwait(0, slots[0])
        gstart(slots[0])
        iload(1, slots[1])

        def body(i, _):
            for b in range(3):
                j = 3 * i + b
                s = slots[b]
                s1 = slots[(b + 1) % 3]
                s2 = slots[(b + 2) % 3]

                @pl.when(j >= 2)
                def _():
                    swait(s1)

                @pl.when(j + 1 < nchunk)
                def _():
                    iwait(j + 1, s1)
                    gstart(s1)

                gwait(s)
                dcopy(s)
                sstart(s)

                @pl.when(j + 2 < nchunk)
                def _():
                    iload(j + 2, s2)

            return 0

        lax.fori_loop(0, nchunk // 3, body, 0)
        swait(slots[1])
        swait(slots[2])
        plsc.subcore_barrier()

        @pl.when(cid == 0)
        def _():
            pltpu.sync_copy(acc.at[pl.ds(sid * RPT, RPT)],
                            out0.at[pl.ds(sid * RPT, RPT)])

        @pl.when(cid == 1)
        def _():
            pltpu.sync_copy(acc.at[pl.ds(sid * RPT, RPT)],
                            out1.at[pl.ds(sid * RPT, RPT)])

    return agg_kernel(h, src, dst)


_BR = 1000


def _tc_linear(x, W, b):
    def body(x_ref, w_ref, b_ref, o_ref):
        o_ref[...] = (
            jnp.dot(x_ref[...], w_ref[...], preferred_element_type=jnp.float32)
            + b_ref[...]
        )

    return pl.pallas_call(
        body,
        grid=(N // _BR,),
        in_specs=[
            pl.BlockSpec((_BR, D), lambda i: (i, 0)),
            pl.BlockSpec((D, D), lambda i: (0, 0)),
            pl.BlockSpec((1, D), lambda i: (0, 0)),
        ],
        out_specs=pl.BlockSpec((_BR, D), lambda i: (i, 0)),
        out_shape=jax.ShapeDtypeStruct((N, D), jnp.float32),
    )(x, W, b.reshape(1, D))


def _tc_add_relu_linear(p0, p1, W, b):
    def body(p0_ref, p1_ref, w_ref, b_ref, o_ref):
        hloc = jnp.maximum(p0_ref[...] + p1_ref[...], 0.0)
        o_ref[...] = (
            jnp.dot(hloc, w_ref[...], preferred_element_type=jnp.float32)
            + b_ref[...]
        )

    return pl.pallas_call(
        body,
        grid=(N // _BR,),
        in_specs=[
            pl.BlockSpec((_BR, D), lambda i: (i, 0)),
            pl.BlockSpec((_BR, D), lambda i: (i, 0)),
            pl.BlockSpec((D, D), lambda i: (0, 0)),
            pl.BlockSpec((1, D), lambda i: (0, 0)),
        ],
        out_specs=pl.BlockSpec((_BR, D), lambda i: (i, 0)),
        out_shape=jax.ShapeDtypeStruct((N, D), jnp.float32),
    )(p0, p1, W, b.reshape(1, D))


def _tc_add_relu_norm(p0, p1):
    def body(p0_ref, p1_ref, o_ref):
        y = jnp.maximum(p0_ref[...] + p1_ref[...], 0.0)
        nrm = jnp.sqrt(jnp.sum(y * y, axis=-1, keepdims=True))
        o_ref[...] = y / jnp.maximum(nrm, 1e-12)

    return pl.pallas_call(
        body,
        grid=(N // _BR,),
        in_specs=[
            pl.BlockSpec((_BR, D), lambda i: (i, 0)),
            pl.BlockSpec((_BR, D), lambda i: (i, 0)),
        ],
        out_specs=pl.BlockSpec((_BR, D), lambda i: (i, 0)),
        out_shape=jax.ShapeDtypeStruct((N, D), jnp.float32),
    )(p0, p1)


def kernel(x, edge_index, W0, b0, W1, b1):
    pad = EP - E
    src = jnp.concatenate([edge_index[0], jnp.zeros((pad,), jnp.int32)])
    dst = jnp.concatenate([edge_index[1], jnp.full((pad,), NP - 1, jnp.int32)])
    h1 = _tc_linear(x, W0, b0)
    a0, a1 = _sc_aggregate(h1, src, dst)
    h2 = _tc_add_relu_linear(a0, a1, W1, b1)
    c0, c1 = _sc_aggregate(h2, src, dst)
    return _tc_add_relu_norm(c0, c1)


# speedup vs baseline: 1.0609x; 1.0609x over previous
import functools

import jax
import jax.numpy as jnp
from jax import lax
from jax.experimental import pallas as pl
from jax.experimental.pallas import tpu as pltpu
from jax.experimental.pallas import tpu_sc as plsc

N = 10000
D = 128
E = 320000
NC = 2
NS = 16
NW = NC * NS
CH = 80
NCHUNK0 = 168
NCHUNK1 = 84
EPW0 = NCHUNK0 * CH
EPW1 = NCHUNK1 * CH
EP = (EPW0 + EPW1) * NS
NP = 10240
RPT = NP // NS
ZCH = CH


def _sc_aggregate(h, src, dst):
    mesh = plsc.VectorSubcoreMesh(core_axis_name="c", subcore_axis_name="s")

    @functools.partial(
        pl.kernel,
        mesh=mesh,
        out_type=[
            jax.ShapeDtypeStruct((NP, D), jnp.float32),
            jax.ShapeDtypeStruct((NP, D), jnp.float32),
        ],
        scratch_types=[
            pltpu.VMEM((CH,), jnp.int32),
            pltpu.VMEM((CH,), jnp.int32),
            pltpu.VMEM((CH,), jnp.int32),
            pltpu.VMEM((CH,), jnp.int32),
            pltpu.VMEM((CH,), jnp.int32),
            pltpu.VMEM((CH,), jnp.int32),
            pltpu.VMEM((CH,), jnp.int32),
            pltpu.VMEM((CH,), jnp.int32),
            pltpu.VMEM((CH,), jnp.int32),
            pltpu.VMEM((CH, D), jnp.float32),
            pltpu.VMEM((CH, D), jnp.float32),
            pltpu.VMEM((CH, D), jnp.float32),
            pltpu.VMEM_SHARED((NP, D), jnp.float32),
            pltpu.SemaphoreType.DMA,
            pltpu.SemaphoreType.DMA,
            pltpu.SemaphoreType.DMA,
            pltpu.SemaphoreType.DMA,
            pltpu.SemaphoreType.DMA,
            pltpu.SemaphoreType.DMA,
            pltpu.SemaphoreType.DMA,
            pltpu.SemaphoreType.DMA,
            pltpu.SemaphoreType.DMA,
            pltpu.SemaphoreType.DMA,
            pltpu.SemaphoreType.DMA,
            pltpu.SemaphoreType.DMA,
        ],
    )
    def agg_kernel(h_hbm, src_hbm, dst_hbm, out0, out1,
                   sidx0, sidx1, sidx2, didx0, didx1, didx2,
                   dcp0, dcp1, dcp2, rows0, rows1, rows2, acc,
                   gsem0, gsem1, gsem2, ssem0, ssem1, ssem2,
                   sisem0, sisem1, sisem2, disem0, disem1, disem2):
        cid = lax.axis_index("c")
        sid = lax.axis_index("s")
        wid = cid * NS + sid

        z = jnp.zeros((16,), jnp.float32)

        def zstore(i, _):
            r = i // (D // 16)
            k = i % (D // 16)
            rows0[r, pl.ds(k * 16, 16)] = z
            return 0

        lax.fori_loop(0, ZCH * (D // 16), zstore, 0)

        def zcopy(j, _):
            pltpu.sync_copy(rows0, acc.at[pl.ds(sid * RPT + j * ZCH, ZCH)])
            return 0

        lax.fori_loop(0, RPT // ZCH, zcopy, 0)
        plsc.subcore_barrier()

        nchunk = jnp.where(cid == 0, NCHUNK0, NCHUNK1)
        base0 = jnp.where(cid == 0, sid * EPW0, NS * EPW0 + sid * EPW1)
        slots = (
            (sidx0, didx0, dcp0, rows0, gsem0, ssem0, sisem0, disem0),
            (sidx1, didx1, dcp1, rows1, gsem1, ssem1, sisem1, disem1),
            (sidx2, didx2, dcp2, rows2, gsem2, ssem2, sisem2, disem2),
        )

        def iload(j, s):
            si, di = s[0], s[1]
            pltpu.async_copy(src_hbm.at[pl.ds(base0 + j * CH, CH)], si, s[6])
            pltpu.async_copy(dst_hbm.at[pl.ds(base0 + j * CH, CH)], di, s[7])

        def iwait(j, s):
            si, di = s[0], s[1]
            pltpu.make_async_copy(
                src_hbm.at[pl.ds(base0 + j * CH, CH)], si, s[6]).wait()
            pltpu.make_async_copy(
                dst_hbm.at[pl.ds(base0 + j * CH, CH)], di, s[7]).wait()

        def gstart(s):
            pltpu.async_copy(h_hbm.at[s[0]], s[3], s[4])

        def gwait(s):
            pltpu.make_async_copy(h_hbm.at[s[0]], s[3], s[4]).wait()

        def sstart(s):
            pltpu.async_copy(s[3], acc.at[s[2]], s[5], add=True)

        def swait(s):
            pltpu.make_async_copy(s[3], acc.at[s[2]], s[5]).wait()

        def dcopy(s):
            di, dc = s[1], s[2]
            for k in range(CH // 16):
                dc[pl.ds(k * 16, 16)] = di[pl.ds(k * 16, 16)]

        iload(0, slots[0])
        iwait(0, slots[0])
        gstart(slots[0])
        iload(1, slots[1])

        def body(i, _):
            for b in range(3):
                j = 3 * i + b
                s = slots[b]
                s1 = slots[(b + 1) % 3]
                s2 = slots[(b + 2) % 3]

                @pl.when(j >= 2)
                def _():
                    swait(s1)

                @pl.when(j + 1 < nchunk)
                def _():
                    iwait(j + 1, s1)
                    gstart(s1)

                gwait(s)
                dcopy(s)
                sstart(s)

                @pl.when(j + 2 < nchunk)
                def _():
                    iload(j + 2, s2)

            return 0

        lax.fori_loop(0, nchunk // 3, body, 0)
        swait(slots[1])
        swait(slots[2])
        plsc.subcore_barrier()

        @pl.when(cid == 0)
        def _():
            pltpu.sync_copy(acc.at[pl.ds(sid * RPT, RPT)],
                            out0.at[pl.ds(sid * RPT, RPT)])

        @pl.when(cid == 1)
        def _():
            pltpu.sync_copy(acc.at[pl.ds(sid * RPT, RPT)],
                            out1.at[pl.ds(sid * RPT, RPT)])

    return agg_kernel(h, src, dst)


_BR = 1000


def _tc_linear(x, W, b):
    def body(x_ref, w_ref, b_ref, o_ref):
        o_ref[...] = (
            jnp.dot(x_ref[...], w_ref[...], preferred_element_type=jnp.float32)
            + b_ref[...]
        )

    return pl.pallas_call(
        body,
        grid=(N // _BR,),
        in_specs=[
            pl.BlockSpec((_BR, D), lambda i: (i, 0)),
            pl.BlockSpec((D, D), lambda i: (0, 0)),
            pl.BlockSpec((1, D), lambda i: (0, 0)),
        ],
        out_specs=pl.BlockSpec((_BR, D), lambda i: (i, 0)),
        out_shape=jax.ShapeDtypeStruct((N, D), jnp.float32),
    )(x, W, b.reshape(1, D))


def _tc_add_relu_linear(p0, p1, W, b):
    def body(p0_ref, p1_ref, w_ref, b_ref, o_ref):
        hloc = jnp.maximum(p0_ref[...] + p1_ref[...], 0.0)
        o_ref[...] = (
            jnp.dot(hloc, w_ref[...], preferred_element_type=jnp.float32)
            + b_ref[...]
        )

    return pl.pallas_call(
        body,
        grid=(N // _BR,),
        in_specs=[
            pl.BlockSpec((_BR, D), lambda i: (i, 0)),
            pl.BlockSpec((_BR, D), lambda i: (i, 0)),
            pl.BlockSpec((D, D), lambda i: (0, 0)),
            pl.BlockSpec((1, D), lambda i: (0, 0)),
        ],
        out_specs=pl.BlockSpec((_BR, D), lambda i: (i, 0)),
        out_shape=jax.ShapeDtypeStruct((N, D), jnp.float32),
    )(p0, p1, W, b.reshape(1, D))


def _tc_add_relu_norm(p0, p1):
    def body(p0_ref, p1_ref, o_ref):
        y = jnp.maximum(p0_ref[...] + p1_ref[...], 0.0)
        nrm = jnp.sqrt(jnp.sum(y * y, axis=-1, keepdims=True))
        o_ref[...] = y / jnp.maximum(nrm, 1e-12)

    return pl.pallas_call(
        body,
        grid=(N // _BR,),
        in_specs=[
            pl.BlockSpec((_BR, D), lambda i: (i, 0)),
            pl.BlockSpec((_BR, D), lambda i: (i, 0)),
        ],
        out_specs=pl.BlockSpec((_BR, D), lambda i: (i, 0)),
        out_shape=jax.ShapeDtypeStruct((N, D), jnp.float32),
    )(p0, p1)


def kernel(x, edge_index, W0, b0, W1, b1):
    pad = EP - E
    src = jnp.concatenate([edge_index[0], jnp.zeros((pad,), jnp.int32)])
    dst = jnp.concatenate([edge_index[1], jnp.full((pad,), NP - 1, jnp.int32)])
    h1 = _tc_linear(x, W0, b0)
    a0, a1 = _sc_aggregate(h1, src, dst)
    h2 = _tc_add_relu_linear(a0, a1, W1, b1)
    c0, c1 = _sc_aggregate(h2, src, dst)
    return _tc_add_relu_norm(c0, c1)


# speedup vs baseline: 1.0948x; 1.0320x over previous
import functools

import jax
import jax.numpy as jnp
from jax import lax
from jax.experimental import pallas as pl
from jax.experimental.pallas import tpu as pltpu
from jax.experimental.pallas import tpu_sc as plsc

N = 10000
D = 128
E = 320000
NC = 2
NS = 16
NW = NC * NS
CH = 80
NCHUNK0 = 180
NCHUNK1 = 72
EPW0 = NCHUNK0 * CH
EPW1 = NCHUNK1 * CH
EP = (EPW0 + EPW1) * NS
NP = 10240
RPT = NP // NS
ZCH = CH


def _sc_aggregate(h, src, dst):
    mesh = plsc.VectorSubcoreMesh(core_axis_name="c", subcore_axis_name="s")

    @functools.partial(
        pl.kernel,
        mesh=mesh,
        out_type=[
            jax.ShapeDtypeStruct((NP, D), jnp.float32),
            jax.ShapeDtypeStruct((NP, D), jnp.float32),
        ],
        scratch_types=[
            pltpu.VMEM((CH,), jnp.int32),
            pltpu.VMEM((CH,), jnp.int32),
            pltpu.VMEM((CH,), jnp.int32),
            pltpu.VMEM((CH,), jnp.int32),
            pltpu.VMEM((CH,), jnp.int32),
            pltpu.VMEM((CH,), jnp.int32),
            pltpu.VMEM((CH,), jnp.int32),
            pltpu.VMEM((CH,), jnp.int32),
            pltpu.VMEM((CH,), jnp.int32),
            pltpu.VMEM((CH, D), jnp.float32),
            pltpu.VMEM((CH, D), jnp.float32),
            pltpu.VMEM((CH, D), jnp.float32),
            pltpu.VMEM_SHARED((NP, D), jnp.float32),
            pltpu.SemaphoreType.DMA,
            pltpu.SemaphoreType.DMA,
            pltpu.SemaphoreType.DMA,
            pltpu.SemaphoreType.DMA,
            pltpu.SemaphoreType.DMA,
            pltpu.SemaphoreType.DMA,
            pltpu.SemaphoreType.DMA,
            pltpu.SemaphoreType.DMA,
            pltpu.SemaphoreType.DMA,
            pltpu.SemaphoreType.DMA,
            pltpu.SemaphoreType.DMA,
            pltpu.SemaphoreType.DMA,
        ],
    )
    def agg_kernel(h_hbm, src_hbm, dst_hbm, out0, out1,
                   sidx0, sidx1, sidx2, didx0, didx1, didx2,
                   dcp0, dcp1, dcp2, rows0, rows1, rows2, acc,
                   gsem0, gsem1, gsem2, ssem0, ssem1, ssem2,
                   sisem0, sisem1, sisem2, disem0, disem1, disem2):
        cid = lax.axis_index("c")
        sid = lax.axis_index("s")
        wid = cid * NS + sid

        z = jnp.zeros((16,), jnp.float32)

        def zstore(i, _):
            r = i // (D // 16)
            k = i % (D // 16)
            rows0[r, pl.ds(k * 16, 16)] = z
            return 0

        lax.fori_loop(0, ZCH * (D // 16), zstore, 0)

        def zcopy(j, _):
            pltpu.sync_copy(rows0, acc.at[pl.ds(sid * RPT + j * ZCH, ZCH)])
            return 0

        lax.fori_loop(0, RPT // ZCH, zcopy, 0)
        plsc.subcore_barrier()

        nchunk = jnp.where(cid == 0, NCHUNK0, NCHUNK1)
        base0 = jnp.where(cid == 0, sid * EPW0, NS * EPW0 + sid * EPW1)
        slots = (
            (sidx0, didx0, dcp0, rows0, gsem0, ssem0, sisem0, disem0),
            (sidx1, didx1, dcp1, rows1, gsem1, ssem1, sisem1, disem1),
            (sidx2, didx2, dcp2, rows2, gsem2, ssem2, sisem2, disem2),
        )

        def iload(j, s):
            si, di = s[0], s[1]
            pltpu.async_copy(src_hbm.at[pl.ds(base0 + j * CH, CH)], si, s[6])
            pltpu.async_copy(dst_hbm.at[pl.ds(base0 + j * CH, CH)], di, s[7])

        def iwait(j, s):
            si, di = s[0], s[1]
            pltpu.make_async_copy(
                src_hbm.at[pl.ds(base0 + j * CH, CH)], si, s[6]).wait()
            pltpu.make_async_copy(
                dst_hbm.at[pl.ds(base0 + j * CH, CH)], di, s[7]).wait()

        def gstart(s):
            pltpu.async_copy(h_hbm.at[s[0]], s[3], s[4])

        def gwait(s):
            pltpu.make_async_copy(h_hbm.at[s[0]], s[3], s[4]).wait()

        def sstart(s):
            pltpu.async_copy(s[3], acc.at[s[2]], s[5], add=True)

        def swait(s):
            pltpu.make_async_copy(s[3], acc.at[s[2]], s[5]).wait()

        def dcopy(s):
            di, dc = s[1], s[2]
            for k in range(CH // 16):
                dc[pl.ds(k * 16, 16)] = di[pl.ds(k * 16, 16)]

        iload(0, slots[0])
        iwait(0, slots[0])
        gstart(slots[0])
        iload(1, slots[1])

        def body(i, _):
            for b in range(3):
                j = 3 * i + b
                s = slots[b]
                s1 = slots[(b + 1) % 3]
                s2 = slots[(b + 2) % 3]

                @pl.when(j >= 2)
                def _():
                    swait(s1)

                @pl.when(j + 1 < nchunk)
                def _():
                    iwait(j + 1, s1)
                    gstart(s1)

                gwait(s)
                dcopy(s)
                sstart(s)

                @pl.when(j + 2 < nchunk)
                def _():
                    iload(j + 2, s2)

            return 0

        lax.fori_loop(0, nchunk // 3, body, 0)
        swait(slots[1])
        swait(slots[2])
        plsc.subcore_barrier()

        @pl.when(cid == 0)
        def _():
            pltpu.sync_copy(acc.at[pl.ds(sid * RPT, RPT)],
                            out0.at[pl.ds(sid * RPT, RPT)])

        @pl.when(cid == 1)
        def _():
            pltpu.sync_copy(acc.at[pl.ds(sid * RPT, RPT)],
                            out1.at[pl.ds(sid * RPT, RPT)])

    return agg_kernel(h, src, dst)


_BR = 1000


def _tc_linear(x, W, b):
    def body(x_ref, w_ref, b_ref, o_ref):
        o_ref[...] = (
            jnp.dot(x_ref[...], w_ref[...], preferred_element_type=jnp.float32)
            + b_ref[...]
        )

    return pl.pallas_call(
        body,
        grid=(N // _BR,),
        in_specs=[
            pl.BlockSpec((_BR, D), lambda i: (i, 0)),
            pl.BlockSpec((D, D), lambda i: (0, 0)),
            pl.BlockSpec((1, D), lambda i: (0, 0)),
        ],
        out_specs=pl.BlockSpec((_BR, D), lambda i: (i, 0)),
        out_shape=jax.ShapeDtypeStruct((N, D), jnp.float32),
    )(x, W, b.reshape(1, D))


def _tc_add_relu_linear(p0, p1, W, b):
    def body(p0_ref, p1_ref, w_ref, b_ref, o_ref):
        hloc = jnp.maximum(p0_ref[...] + p1_ref[...], 0.0)
        o_ref[...] = (
            jnp.dot(hloc, w_ref[...], preferred_element_type=jnp.float32)
            + b_ref[...]
        )

    return pl.pallas_call(
        body,
        grid=(N // _BR,),
        in_specs=[
            pl.BlockSpec((_BR, D), lambda i: (i, 0)),
            pl.BlockSpec((_BR, D), lambda i: (i, 0)),
            pl.BlockSpec((D, D), lambda i: (0, 0)),
            pl.BlockSpec((1, D), lambda i: (0, 0)),
        ],
        out_specs=pl.BlockSpec((_BR, D), lambda i: (i, 0)),
        out_shape=jax.ShapeDtypeStruct((N, D), jnp.float32),
    )(p0, p1, W, b.reshape(1, D))


def _tc_add_relu_norm(p0, p1):
    def body(p0_ref, p1_ref, o_ref):
        y = jnp.maximum(p0_ref[...] + p1_ref[...], 0.0)
        nrm = jnp.sqrt(jnp.sum(y * y, axis=-1, keepdims=True))
        o_ref[...] = y / jnp.maximum(nrm, 1e-12)

    return pl.pallas_call(
        body,
        grid=(N // _BR,),
        in_specs=[
            pl.BlockSpec((_BR, D), lambda i: (i, 0)),
            pl.BlockSpec((_BR, D), lambda i: (i, 0)),
        ],
        out_specs=pl.BlockSpec((_BR, D), lambda i: (i, 0)),
        out_shape=jax.ShapeDtypeStruct((N, D), jnp.float32),
    )(p0, p1)


def kernel(x, edge_index, W0, b0, W1, b1):
    pad = EP - E
    src = jnp.concatenate([edge_index[0], jnp.zeros((pad,), jnp.int32)])
    dst = jnp.concatenate([edge_index[1], jnp.full((pad,), NP - 1, jnp.int32)])
    h1 = _tc_linear(x, W0, b0)
    a0, a1 = _sc_aggregate(h1, src, dst)
    h2 = _tc_add_relu_linear(a0, a1, W1, b1)
    c0, c1 = _sc_aggregate(h2, src, dst)
    return _tc_add_relu_norm(c0, c1)
